# baseline (speedup 1.0000x reference)
import functools

import jax
import jax.numpy as jnp
from jax import lax
from jax.experimental import pallas as pl
from jax.experimental.pallas import tpu as pltpu
from jax.experimental.pallas import tpu_sc as plsc

_NC = 2
_NS = 16


@functools.lru_cache(maxsize=None)
def _make_gather(V, D, DP, Bt, F):
    NW = _NC * _NS
    assert Bt % NW == 0, (Bt, NW)
    b_per_w = Bt // NW
    nbuf = 8
    mesh = plsc.VectorSubcoreMesh(core_axis_name="c", subcore_axis_name="s")

    @functools.partial(
        pl.kernel,
        mesh=mesh,
        out_type=jax.ShapeDtypeStruct((Bt, F, D), jnp.float32),
        scratch_types=(
            [pltpu.VMEM((b_per_w, F), jnp.int32)]
            + [pltpu.VMEM((F, DP), jnp.float32) for _ in range(nbuf)]
            + [pltpu.SemaphoreType.DMA for _ in range(nbuf)]
        ),
        compiler_params=pltpu.CompilerParams(use_tc_tiling_on_sc=False),
    )
    def gather_kernel(x_hbm, table_hbm, out_hbm, idx_v, *rest):
        bufs, sems = rest[:nbuf], rest[nbuf:]
        wid = lax.axis_index("s") * _NC + lax.axis_index("c")
        b_base = wid * b_per_w
        pltpu.sync_copy(x_hbm.at[pl.ds(b_base, b_per_w)], idx_v)

        def fire(c, slot):
            return pltpu.async_copy(
                table_hbm.at[idx_v.at[c]], bufs[slot], sems[slot]
            )

        handles = [fire(c, c) for c in range(nbuf)]
        for c in range(b_per_w):
            slot = c % nbuf
            handles[slot].wait()
            pltpu.sync_copy(
                bufs[slot].at[:, pl.ds(0, D)], out_hbm.at[b_base + c]
            )
            nc = c + nbuf
            if nc < b_per_w:
                handles[slot] = fire(nc, slot)

    return gather_kernel


def kernel(x, weight):
    Bt, F = x.shape
    V, D = weight.shape
    DP = 80
    wp = jnp.pad(weight, ((0, 0), (0, DP - D)))
    return _make_gather(V, D, DP, Bt, F)(x.astype(jnp.int32), wp)


# speedup vs baseline: 1.8310x; 1.8310x over previous
import functools

import jax
import jax.numpy as jnp
from jax import lax
from jax.experimental import pallas as pl
from jax.experimental.pallas import tpu as pltpu
from jax.experimental.pallas import tpu_sc as plsc

_NC = 2
_NS = 16


@functools.lru_cache(maxsize=None)
def _make_gather(V, D, DP, Bt, F):
    NW = _NC * _NS
    assert Bt % NW == 0, (Bt, NW)
    b_per_w = Bt // NW
    nbuf = 16
    mesh = plsc.VectorSubcoreMesh(core_axis_name="c", subcore_axis_name="s")

    @functools.partial(
        pl.kernel,
        mesh=mesh,
        out_type=jax.ShapeDtypeStruct((Bt, F, D), jnp.float32),
        scratch_types=(
            [pltpu.VMEM((b_per_w, F), jnp.int32)]
            + [pltpu.VMEM((F, DP), jnp.float32) for _ in range(nbuf)]
            + [pltpu.SemaphoreType.DMA for _ in range(nbuf)]
        ),
        compiler_params=pltpu.CompilerParams(use_tc_tiling_on_sc=False),
    )
    def gather_kernel(x_hbm, table_hbm, out_hbm, idx_v, *rest):
        bufs, sems = rest[:nbuf], rest[nbuf:]
        wid = lax.axis_index("s") * _NC + lax.axis_index("c")
        b_base = wid * b_per_w
        pltpu.sync_copy(x_hbm.at[pl.ds(b_base, b_per_w)], idx_v)

        def fire(c, slot):
            return pltpu.async_copy(
                table_hbm.at[idx_v.at[c]], bufs[slot], sems[slot]
            )

        handles = [fire(c, c) for c in range(nbuf)]
        for c in range(b_per_w):
            slot = c % nbuf
            handles[slot].wait()
            pltpu.sync_copy(
                bufs[slot].at[:, pl.ds(0, D)], out_hbm.at[b_base + c]
            )
            nc = c + nbuf
            if nc < b_per_w:
                handles[slot] = fire(nc, slot)

    return gather_kernel


def kernel(x, weight):
    Bt, F = x.shape
    V, D = weight.shape
    DP = 128
    wp = jnp.pad(weight, ((0, 0), (0, DP - D)))
    return _make_gather(V, D, DP, Bt, F)(x.astype(jnp.int32), wp)
